# baseline (speedup 1.0000x reference)
import functools

import jax
import jax.numpy as jnp
from jax import lax
from jax.experimental import pallas as pl
from jax.experimental.pallas import tpu as pltpu
from jax.experimental.pallas import tpu_sc as plsc

BN_EPS = 1e-5
L = 16
NTILES = 16
NCORES = 2
NW = NTILES * NCORES
CHUNK = 128


DW = 16


def _make_deg(cw, npad):
    rpt = npad // NTILES

    @functools.partial(
        pl.kernel,
        out_type=jax.ShapeDtypeStruct((NCORES, npad, DW), jnp.float32),
        mesh=plsc.VectorSubcoreMesh(core_axis_name="c", subcore_axis_name="s"),
        scratch_types=[
            pltpu.VMEM((4, CHUNK), jnp.int32),
            pltpu.VMEM((CHUNK, DW), jnp.float32),
            pltpu.VMEM_SHARED((npad, DW), jnp.float32),
        ] + [pltpu.SemaphoreType.DMA] * 6,
    )
    def deg_kernel(dst_hbm, z8_hbm, o8_hbm, out_hbm, didx_v, ones_v, acc_sh,
                   *sems):
        idsem = sems[0:4]
        ssem = sems[4:6]
        cid = lax.axis_index("c")
        sid = lax.axis_index("s")
        wid = cid * NTILES + sid
        base = wid * cw
        row0 = sid * rpt

        def didx_load(c, slot):
            return pltpu.make_async_copy(
                dst_hbm.at[pl.ds((base + c) * CHUNK, CHUNK)],
                didx_v.at[slot], idsem[slot])

        def scatter(slot, q):
            return pltpu.make_async_copy(
                ones_v, acc_sh.at[didx_v.at[slot]], ssem[q])

        for slot in range(2):
            didx_load(slot, slot).start()
        pltpu.sync_copy(o8_hbm, ones_v)
        pltpu.sync_copy(z8_hbm.at[pl.ds(row0, rpt)],
                        acc_sh.at[pl.ds(row0, rpt)])
        plsc.subcore_barrier()

        def grp(gi, carry):
            for j in range(4):
                q = j % 2
                c = gi * 4 + j
                didx_load(c, j).wait()

                @pl.when(c >= 2)
                def _():
                    scatter((j + 2) % 4, q).wait()

                pltpu.async_copy(ones_v, acc_sh.at[didx_v.at[j]], ssem[q],
                                 add=True)

                @pl.when(c + 2 < cw)
                def _():
                    didx_load(c + 2, (j + 2) % 4).start()
            return carry

        lax.fori_loop(0, cw // 4, grp, 0)
        for i in range(2):
            scatter((cw - 2 + i) % 4, i % 2).wait()
        plsc.subcore_barrier()
        pltpu.sync_copy(acc_sh.at[pl.ds(row0, rpt)],
                        out_hbm.at[cid, pl.ds(row0, rpt)])

    return deg_kernel


def _make_prop(cw, npad, d):
    rpt = npad // NTILES

    @functools.partial(
        pl.kernel,
        out_type=jax.ShapeDtypeStruct((NCORES, npad, d), jnp.float32),
        mesh=plsc.VectorSubcoreMesh(core_axis_name="c", subcore_axis_name="s"),
        scratch_types=[
            pltpu.VMEM((4, CHUNK), jnp.int32),
            pltpu.VMEM((2, CHUNK), jnp.int32),
            pltpu.VMEM((2, CHUNK, d), jnp.float32),
            pltpu.VMEM_SHARED((npad, d), jnp.float32),
        ] + [pltpu.SemaphoreType.DMA] * 8,
    )
    def prop_kernel(m_hbm, src_hbm, dst_hbm, z_hbm, out_hbm,
                    sidx_v, didx_v, rows_v, acc_sh, *sems):
        gsem = sems[0:2]
        issem = sems[2:6]
        idsem = sems[6:8]
        cid = lax.axis_index("c")
        sid = lax.axis_index("s")
        wid = cid * NTILES + sid
        base = wid * cw
        row0 = sid * rpt

        def sidx_load(c, slot):
            return pltpu.make_async_copy(
                src_hbm.at[pl.ds((base + c) * CHUNK, CHUNK)],
                sidx_v.at[slot], issem[slot])

        def didx_load(c, slot):
            return pltpu.make_async_copy(
                dst_hbm.at[pl.ds((base + c) * CHUNK, CHUNK)],
                didx_v.at[slot], idsem[slot])

        def gather(slot, b):
            return pltpu.make_async_copy(
                m_hbm.at[sidx_v.at[slot]], rows_v.at[b], gsem[b])

        for q in range(4):
            sidx_load(q, q).start()
        for b in range(2):
            didx_load(b, b).start()
        pltpu.sync_copy(z_hbm.at[pl.ds(row0, rpt)],
                        acc_sh.at[pl.ds(row0, rpt)])
        plsc.subcore_barrier()
        for b in range(2):
            sidx_load(b, b).wait()
            gather(b, b).start()

        def grp(gi, carry):
            for j in range(4):
                b = j % 2
                c = gi * 4 + j
                gather(j, b).wait()
                didx_load(c, b).wait()
                pltpu.sync_copy(rows_v.at[b], acc_sh.at[didx_v.at[b]],
                                add=True)

                @pl.when(c + 2 < cw)
                def _():
                    didx_load(c + 2, b).start()

                @pl.when(c + 4 < cw)
                def _():
                    sidx_load(c + 4, j).start()

                @pl.when(c + 2 < cw)
                def _():
                    sidx_load(c + 2, (j + 2) % 4).wait()
                    gather((j + 2) % 4, b).start()
            return carry

        lax.fori_loop(0, cw // 4, grp, 0)
        plsc.subcore_barrier()
        pltpu.sync_copy(acc_sh.at[pl.ds(row0, rpt)],
                        out_hbm.at[cid, pl.ds(row0, rpt)])

    return prop_kernel


def _tca_body(x_ref, w1_ref, h_ref):
    h_ref[...] = jnp.dot(x_ref[...], w1_ref[...],
                         preferred_element_type=jnp.float32)


def _tcb_body(degp_ref, h_ref, dinv_ref, m1_ref, *, n):
    deg = degp_ref[0, :, 0:1] + degp_ref[1, :, 0:1] + 1.0
    dinv = lax.rsqrt(deg)
    dinv_ref[...] = dinv
    npad = m1_ref.shape[0]
    m1_ref[0:n, :] = h_ref[...] * dinv[0:n]
    m1_ref[n:npad, :] = jnp.zeros((npad - n, m1_ref.shape[1]), jnp.float32)


def _tc2_body(acc_ref, m_ref, dinv_ref, b_ref, g_ref, be_ref, w_ref,
              out_ref, *, n):
    dinv = dinv_ref[...]
    s = acc_ref[0, 0:n, :] + acc_ref[1, 0:n, :] + m_ref[0:n, :]
    out = s * dinv[0:n] + b_ref[...]
    mean = jnp.mean(out, axis=0, keepdims=True)
    var = jnp.mean((out - mean) ** 2, axis=0, keepdims=True)
    out = (out - mean) * lax.rsqrt(var + BN_EPS) * g_ref[...] + be_ref[...]
    out = jnp.maximum(out, 0.0)
    h = jnp.dot(out, w_ref[...], preferred_element_type=jnp.float32)
    npad = out_ref.shape[0]
    out_ref[0:n, :] = h * dinv[0:n]
    out_ref[n:npad, :] = jnp.zeros((npad - n, h.shape[1]), jnp.float32)


def _tc3_body(acc_ref, m_ref, dinv_ref, b_ref, g_ref, be_ref, wh_ref,
              bh_ref, y_ref, *, n):
    dinv = dinv_ref[...]
    s = acc_ref[0, 0:n, :] + acc_ref[1, 0:n, :] + m_ref[0:n, :]
    out = s * dinv[0:n] + b_ref[...]
    mean = jnp.mean(out, axis=0, keepdims=True)
    var = jnp.mean((out - mean) ** 2, axis=0, keepdims=True)
    out = (out - mean) * lax.rsqrt(var + BN_EPS) * g_ref[...] + be_ref[...]
    out = jnp.maximum(out, 0.0)
    y_ref[...] = (jnp.dot(out, wh_ref[...], preferred_element_type=jnp.float32)
                  + bh_ref[...])


def kernel(x, edge_index, W1, b1, g1, be1, W2, b2, g2, be2, Wh, bh):
    n, _ = x.shape
    hdim = W1.shape[1]
    e = edge_index.shape[1]
    npad = ((n + 1 + 127) // 128) * 128
    group = NW * CHUNK * 8
    epad = ((e + group - 1) // group) * group
    pad = epad - e
    cw = epad // NW // CHUNK

    src = edge_index[0].astype(jnp.int32)
    dst = edge_index[1].astype(jnp.int32)
    if pad:
        pad_idx = n + (jnp.arange(pad, dtype=jnp.int32) % (npad - n))
        src = jnp.concatenate([src, pad_idx])
        dst = jnp.concatenate([dst, pad_idx])

    z8 = jnp.zeros((npad, DW), jnp.float32)
    o8 = jnp.ones((CHUNK, DW), jnp.float32)
    zd = jnp.zeros((npad, hdim), jnp.float32)

    deg_p = _make_deg(cw, npad)(dst, z8, o8)

    h1 = pl.pallas_call(
        _tca_body,
        out_shape=jax.ShapeDtypeStruct((n, hdim), jnp.float32),
    )(x, W1)

    b1r, g1r, be1r = (v.reshape(1, hdim) for v in (b1, g1, be1))
    b2r, g2r, be2r = (v.reshape(1, hdim) for v in (b2, g2, be2))
    bhr = bh.reshape(1, 1)

    dinv, m1 = pl.pallas_call(
        functools.partial(_tcb_body, n=n),
        out_shape=[jax.ShapeDtypeStruct((npad, 1), jnp.float32),
                   jax.ShapeDtypeStruct((npad, hdim), jnp.float32)],
    )(deg_p, h1)

    prop = _make_prop(cw, npad, hdim)
    acc1 = prop(m1, src, dst, zd)

    m2 = pl.pallas_call(
        functools.partial(_tc2_body, n=n),
        out_shape=jax.ShapeDtypeStruct((npad, hdim), jnp.float32),
    )(acc1, m1, dinv, b1r, g1r, be1r, W2)

    acc2 = prop(m2, src, dst, zd)

    y = pl.pallas_call(
        functools.partial(_tc3_body, n=n),
        out_shape=jax.ShapeDtypeStruct((n, 1), jnp.float32),
    )(acc2, m2, dinv, b2r, g2r, be2r, Wh, bhr)

    return y


# speedup vs baseline: 1.0627x; 1.0627x over previous
import functools

import jax
import jax.numpy as jnp
from jax import lax
from jax.experimental import pallas as pl
from jax.experimental.pallas import tpu as pltpu
from jax.experimental.pallas import tpu_sc as plsc

BN_EPS = 1e-5
L = 16
NTILES = 16
NCORES = 2
NW = NTILES * NCORES
CHUNK = 128


def _fill2d(ref, nrows, ncols, val):
    v = jnp.full((L,), val, jnp.float32)

    def body(r, carry):
        for j in range(ncols // L):
            ref[r, pl.ds(j * L, L)] = v
        return carry

    lax.fori_loop(0, nrows, body, 0)


def _zero_acc_slice(zsrc, acc_sh, row0, rpt):
    nfull = rpt // CHUNK
    rem = rpt - nfull * CHUNK
    for j in range(nfull):
        pltpu.sync_copy(zsrc, acc_sh.at[pl.ds(row0 + j * CHUNK, CHUNK)])
    if rem:
        pltpu.sync_copy(zsrc.at[pl.ds(0, rem)],
                        acc_sh.at[pl.ds(row0 + nfull * CHUNK, rem)])


def _make_deg(cw, npad):
    rpt = npad // NTILES

    @functools.partial(
        pl.kernel,
        out_type=jax.ShapeDtypeStruct((NCORES, npad, L), jnp.float32),
        mesh=plsc.VectorSubcoreMesh(core_axis_name="c", subcore_axis_name="s"),
        scratch_types=[
            pltpu.VMEM((4, CHUNK), jnp.int32),
            pltpu.VMEM((CHUNK, L), jnp.float32),
            pltpu.VMEM_SHARED((npad, L), jnp.float32),
        ] + [pltpu.SemaphoreType.DMA] * 6,
    )
    def deg_kernel(dst_hbm, out_hbm, didx_v, ones_v, acc_sh, *sems):
        idsem = sems[0:4]
        ssem = sems[4:6]
        cid = lax.axis_index("c")
        sid = lax.axis_index("s")
        wid = cid * NTILES + sid
        base = wid * cw
        row0 = sid * rpt

        def didx_load(c, slot):
            return pltpu.make_async_copy(
                dst_hbm.at[pl.ds((base + c) * CHUNK, CHUNK)],
                didx_v.at[slot], idsem[slot])

        def scatter(slot, q):
            return pltpu.make_async_copy(
                ones_v, acc_sh.at[didx_v.at[slot]], ssem[q])

        for slot in range(2):
            didx_load(slot, slot).start()
        _fill2d(ones_v, CHUNK, L, 0.0)
        _zero_acc_slice(ones_v, acc_sh, row0, rpt)
        _fill2d(ones_v, CHUNK, L, 1.0)
        plsc.subcore_barrier()

        def grp(gi, carry):
            for j in range(4):
                q = j % 2
                c = gi * 4 + j
                didx_load(c, j).wait()

                @pl.when(c >= 2)
                def _():
                    scatter((j + 2) % 4, q).wait()

                pltpu.async_copy(ones_v, acc_sh.at[didx_v.at[j]], ssem[q],
                                 add=True)

                @pl.when(c + 2 < cw)
                def _():
                    didx_load(c + 2, (j + 2) % 4).start()
            return carry

        lax.fori_loop(0, cw // 4, grp, 0)
        for i in range(2):
            scatter((cw - 2 + i) % 4, i % 2).wait()
        plsc.subcore_barrier()
        pltpu.sync_copy(acc_sh.at[pl.ds(row0, rpt)],
                        out_hbm.at[cid, pl.ds(row0, rpt)])

    return deg_kernel


def _make_prop(cw, npad, d):
    rpt = npad // NTILES

    @functools.partial(
        pl.kernel,
        out_type=jax.ShapeDtypeStruct((NCORES, npad, d), jnp.float32),
        mesh=plsc.VectorSubcoreMesh(core_axis_name="c", subcore_axis_name="s"),
        scratch_types=[
            pltpu.VMEM((4, CHUNK), jnp.int32),
            pltpu.VMEM((2, CHUNK), jnp.int32),
            pltpu.VMEM((2, CHUNK, d), jnp.float32),
            pltpu.VMEM_SHARED((npad, d), jnp.float32),
        ] + [pltpu.SemaphoreType.DMA] * 8,
    )
    def prop_kernel(m_hbm, src_hbm, dst_hbm, out_hbm,
                    sidx_v, didx_v, rows_v, acc_sh, *sems):
        gsem = sems[0:2]
        issem = sems[2:6]
        idsem = sems[6:8]
        cid = lax.axis_index("c")
        sid = lax.axis_index("s")
        wid = cid * NTILES + sid
        base = wid * cw
        row0 = sid * rpt

        def sidx_load(c, slot):
            return pltpu.make_async_copy(
                src_hbm.at[pl.ds((base + c) * CHUNK, CHUNK)],
                sidx_v.at[slot], issem[slot])

        def didx_load(c, slot):
            return pltpu.make_async_copy(
                dst_hbm.at[pl.ds((base + c) * CHUNK, CHUNK)],
                didx_v.at[slot], idsem[slot])

        def gather(slot, b):
            return pltpu.make_async_copy(
                m_hbm.at[sidx_v.at[slot]], rows_v.at[b], gsem[b])

        for q in range(4):
            sidx_load(q, q).start()
        for b in range(2):
            didx_load(b, b).start()
        _fill2d(rows_v.at[0], CHUNK, d, 0.0)
        _zero_acc_slice(rows_v.at[0], acc_sh, row0, rpt)
        plsc.subcore_barrier()
        for b in range(2):
            sidx_load(b, b).wait()
            gather(b, b).start()

        def grp(gi, carry):
            for j in range(4):
                b = j % 2
                c = gi * 4 + j
                gather(j, b).wait()
                didx_load(c, b).wait()
                pltpu.sync_copy(rows_v.at[b], acc_sh.at[didx_v.at[b]],
                                add=True)

                @pl.when(c + 2 < cw)
                def _():
                    didx_load(c + 2, b).start()

                @pl.when(c + 4 < cw)
                def _():
                    sidx_load(c + 4, j).start()

                @pl.when(c + 2 < cw)
                def _():
                    sidx_load(c + 2, (j + 2) % 4).wait()
                    gather((j + 2) % 4, b).start()
            return carry

        lax.fori_loop(0, cw // 4, grp, 0)
        plsc.subcore_barrier()
        pltpu.sync_copy(acc_sh.at[pl.ds(row0, rpt)],
                        out_hbm.at[cid, pl.ds(row0, rpt)])

    return prop_kernel


def _tca_body(x_ref, w1_ref, h_ref):
    h_ref[...] = jnp.dot(x_ref[...], w1_ref[...],
                         preferred_element_type=jnp.float32)


def _tcb_body(degp_ref, h_ref, dinv_ref, m1_ref, *, n):
    deg = degp_ref[0, :, 0:1] + degp_ref[1, :, 0:1] + 1.0
    dinv = lax.rsqrt(deg)
    dinv_ref[...] = dinv
    npad = m1_ref.shape[0]
    m1_ref[0:n, :] = h_ref[...] * dinv[0:n]
    m1_ref[n:npad, :] = jnp.zeros((npad - n, m1_ref.shape[1]), jnp.float32)


def _tc2_body(acc_ref, m_ref, dinv_ref, b_ref, g_ref, be_ref, w_ref,
              out_ref, *, n):
    dinv = dinv_ref[...]
    s = acc_ref[0, 0:n, :] + acc_ref[1, 0:n, :] + m_ref[0:n, :]
    out = s * dinv[0:n] + b_ref[...]
    mean = jnp.mean(out, axis=0, keepdims=True)
    var = jnp.mean((out - mean) ** 2, axis=0, keepdims=True)
    out = (out - mean) * lax.rsqrt(var + BN_EPS) * g_ref[...] + be_ref[...]
    out = jnp.maximum(out, 0.0)
    h = jnp.dot(out, w_ref[...], preferred_element_type=jnp.float32)
    npad = out_ref.shape[0]
    out_ref[0:n, :] = h * dinv[0:n]
    out_ref[n:npad, :] = jnp.zeros((npad - n, h.shape[1]), jnp.float32)


def _tc3_body(acc_ref, m_ref, dinv_ref, b_ref, g_ref, be_ref, wh_ref,
              bh_ref, y_ref, *, n):
    dinv = dinv_ref[...]
    s = acc_ref[0, 0:n, :] + acc_ref[1, 0:n, :] + m_ref[0:n, :]
    out = s * dinv[0:n] + b_ref[...]
    mean = jnp.mean(out, axis=0, keepdims=True)
    var = jnp.mean((out - mean) ** 2, axis=0, keepdims=True)
    out = (out - mean) * lax.rsqrt(var + BN_EPS) * g_ref[...] + be_ref[...]
    out = jnp.maximum(out, 0.0)
    y_ref[...] = (jnp.dot(out, wh_ref[...], preferred_element_type=jnp.float32)
                  + bh_ref[...])


def kernel(x, edge_index, W1, b1, g1, be1, W2, b2, g2, be2, Wh, bh):
    n, _ = x.shape
    hdim = W1.shape[1]
    e = edge_index.shape[1]
    npad = ((n + 1 + 127) // 128) * 128
    group = NW * CHUNK * 4
    epad = ((e + group - 1) // group) * group
    pad = epad - e
    cw = epad // NW // CHUNK

    src = edge_index[0].astype(jnp.int32)
    dst = edge_index[1].astype(jnp.int32)
    if pad:
        pad_idx = n + (jnp.arange(pad, dtype=jnp.int32) % (npad - n))
        src = jnp.concatenate([src, pad_idx])
        dst = jnp.concatenate([dst, pad_idx])

    deg_p = _make_deg(cw, npad)(dst)

    h1 = pl.pallas_call(
        _tca_body,
        out_shape=jax.ShapeDtypeStruct((n, hdim), jnp.float32),
    )(x, W1)

    b1r, g1r, be1r = (v.reshape(1, hdim) for v in (b1, g1, be1))
    b2r, g2r, be2r = (v.reshape(1, hdim) for v in (b2, g2, be2))
    bhr = bh.reshape(1, 1)

    dinv, m1 = pl.pallas_call(
        functools.partial(_tcb_body, n=n),
        out_shape=[jax.ShapeDtypeStruct((npad, 1), jnp.float32),
                   jax.ShapeDtypeStruct((npad, hdim), jnp.float32)],
    )(deg_p, h1)

    prop = _make_prop(cw, npad, hdim)
    acc1 = prop(m1, src, dst)

    m2 = pl.pallas_call(
        functools.partial(_tc2_body, n=n),
        out_shape=jax.ShapeDtypeStruct((npad, hdim), jnp.float32),
    )(acc1, m1, dinv, b1r, g1r, be1r, W2)

    acc2 = prop(m2, src, dst)

    y = pl.pallas_call(
        functools.partial(_tc3_body, n=n),
        out_shape=jax.ShapeDtypeStruct((n, 1), jnp.float32),
    )(acc2, m2, dinv, b2r, g2r, be2r, Wh, bhr)

    return y
